# Initial kernel scaffold; baseline (speedup 1.0000x reference)
import jax
import jax.numpy as jnp
from jax.experimental import pallas as pl


def kernel(x, edge_index, adj, s, pos, W_rel1, b_rel1, W_root1, W_rel2, b_rel2, W_root2, W_rel3, b_rel3, W_root3):
    raise NotImplementedError("write your pallas kernel here")


# speedup vs baseline: 5.0149x; 5.0149x over previous
import functools

import jax
import jax.numpy as jnp
from jax import lax
from jax.experimental import pallas as pl
from jax.experimental.pallas import tpu as pltpu
from jax.experimental.pallas import tpu_sc as plsc

N = 9996
E = 319872
NG = 147
NN = 68
H = 128
C = 16
EPS = 1e-15

NP = 10240
PADROW = NP - 1
NW = 32
K = 128
EW = (E + NW * K - 1) // (NW * K) * K
NCH = EW // K
EPAD = EW * NW
ZR = NP // 16


def _sc_segsum(D):
    mesh = plsc.VectorSubcoreMesh(core_axis_name="c", subcore_axis_name="s")

    @functools.partial(
        pl.kernel,
        out_type=jax.ShapeDtypeStruct((2, NP, D), jnp.float32),
        mesh=mesh,
        compiler_params=pltpu.CompilerParams(use_tc_tiling_on_sc=False),
        scratch_types=[
            pltpu.VMEM((NCH, K), jnp.int32),
            pltpu.VMEM((NCH, K), jnp.int32),
            pltpu.VMEM((K, D), jnp.float32),
            pltpu.VMEM_SHARED((NP, D), jnp.float32),
            pltpu.SemaphoreType.DMA,
        ],
    )
    def k(h_hbm, srcr_hbm, dstr_hbm, zeros_hbm, out_hbm,
          src_v, dst_v, rows_v, acc_sh, sem):
        c = lax.axis_index("c")
        sid = lax.axis_index("s")
        w = sid * 2 + c
        pltpu.sync_copy(srcr_hbm.at[w], src_v)
        pltpu.sync_copy(dstr_hbm.at[w], dst_v)
        pltpu.sync_copy(zeros_hbm.at[pl.ds(sid * ZR, ZR)],
                        acc_sh.at[pl.ds(sid * ZR, ZR)])
        plsc.subcore_barrier()

        @pl.loop(0, NCH)
        def _(j):
            pltpu.async_copy(h_hbm.at[src_v.at[j]], rows_v, sem).wait()
            pltpu.sync_copy(rows_v, acc_sh.at[dst_v.at[j]], add=True)

        plsc.subcore_barrier()
        pltpu.sync_copy(acc_sh.at[pl.ds(sid * ZR, ZR)],
                        out_hbm.at[c, pl.ds(sid * ZR, ZR)])

    return k


def _tc_layer(p, h_prev, A, Br, bias):
    Dp = p.shape[2]
    Din = h_prev.shape[1]
    BRW = 512

    def body(p_ref, h_ref, a_ref, b_ref, bias_ref, o_ref):
        agg = p_ref[0] + p_ref[1]
        acc = jnp.dot(agg, a_ref[...], preferred_element_type=jnp.float32)
        acc = acc + jnp.dot(h_ref[...], b_ref[...],
                            preferred_element_type=jnp.float32)
        acc = acc + bias_ref[...]
        i = pl.program_id(0)
        rows = i * BRW + lax.broadcasted_iota(jnp.int32, (BRW, 1), 0)
        o_ref[...] = jnp.where(rows < N, jnp.maximum(acc, 0.0), 0.0)

    return pl.pallas_call(
        body,
        grid=(NP // BRW,),
        in_specs=[
            pl.BlockSpec((2, BRW, Dp), lambda i: (0, i, 0)),
            pl.BlockSpec((BRW, Din), lambda i: (i, 0)),
            pl.BlockSpec((Dp, H), lambda i: (0, 0)),
            pl.BlockSpec((Din, H), lambda i: (0, 0)),
            pl.BlockSpec((1, H), lambda i: (0, 0)),
        ],
        out_specs=pl.BlockSpec((BRW, H), lambda i: (i, 0)),
        out_shape=jax.ShapeDtypeStruct((NP, H), jnp.float32),
    )(p, h_prev, A, Br, bias)


BG = 7


def _tc_pool(xr, adj_p, s_p):
    def body(xr_ref, adj_ref, s_ref, out_ref, oadj_ref, acc_ref):
        g = pl.program_id(0)

        @pl.when(g == 0)
        def _():
            acc_ref[0, 0] = 0.0
            acc_ref[0, 1] = 0.0

        link_tot = jnp.float32(0.0)
        ent_tot = jnp.float32(0.0)
        rows = lax.broadcasted_iota(jnp.int32, (128, 1), 0)
        for t in range(BG):
            sg = s_ref[t]
            m = jnp.exp(sg - jnp.max(sg, axis=-1, keepdims=True))
            ssm = m / jnp.sum(m, axis=-1, keepdims=True)
            ssm = jnp.where(rows < NN, ssm, 0.0)
            xg = xr_ref[t]
            ag = adj_ref[t]
            out_ref[t] = lax.dot_general(
                ssm, xg, (((0,), (0,)), ((), ())),
                preferred_element_type=jnp.float32)
            ta = lax.dot_general(
                ssm, ag, (((0,), (0,)), ((), ())),
                preferred_element_type=jnp.float32)
            oadj_ref[t] = lax.dot_general(
                ta, ssm, (((1,), (0,)), ((), ())),
                preferred_element_type=jnp.float32)
            link = ag - lax.dot_general(
                ssm, ssm, (((1,), (1,)), ((), ())),
                preferred_element_type=jnp.float32)
            link_tot = link_tot + jnp.sum(link * link)
            ent_tot = ent_tot + jnp.sum(-ssm * jnp.log(ssm + EPS))
        acc_ref[0, 0] += link_tot
        acc_ref[0, 1] += ent_tot

    return pl.pallas_call(
        body,
        grid=(NG // BG,),
        in_specs=[
            pl.BlockSpec((BG, 128, H), lambda g: (g, 0, 0)),
            pl.BlockSpec((BG, 128, 128), lambda g: (g, 0, 0)),
            pl.BlockSpec((BG, 128, C), lambda g: (g, 0, 0)),
        ],
        out_specs=[
            pl.BlockSpec((BG, C, H), lambda g: (g, 0, 0)),
            pl.BlockSpec((BG, C, C), lambda g: (g, 0, 0)),
            pl.BlockSpec(memory_space=pltpu.SMEM),
        ],
        out_shape=[
            jax.ShapeDtypeStruct((NG, C, H), jnp.float32),
            jax.ShapeDtypeStruct((NG, C, C), jnp.float32),
            jax.ShapeDtypeStruct((1, 2), jnp.float32),
        ],
    )(xr, adj_p, s_p)


def kernel(x, edge_index, adj, s, pos,
           W_rel1, b_rel1, W_root1,
           W_rel2, b_rel2, W_root2,
           W_rel3, b_rel3, W_root3):
    src = edge_index[0]
    dst = edge_index[1]
    padn = EPAD - E
    srcr = jnp.concatenate(
        [src, jnp.full((padn,), PADROW, jnp.int32)]).reshape(NW, NCH, K)
    dstr = jnp.concatenate(
        [dst, jnp.full((padn,), PADROW, jnp.int32)]).reshape(NW, NCH, K)

    x16 = jnp.zeros((NP, 16), jnp.float32).at[:N, :2].set(x)
    z16 = jnp.zeros((NP, 16), jnp.float32)
    z128 = jnp.zeros((NP, H), jnp.float32)

    A1 = jnp.zeros((16, H), jnp.float32).at[:2, :].set(W_rel1.T)
    B1 = jnp.zeros((16, H), jnp.float32).at[:2, :].set(W_root1.T)

    seg16 = _sc_segsum(16)
    seg128 = _sc_segsum(H)

    p1 = seg16(x16, srcr, dstr, z16)
    h1 = _tc_layer(p1, x16, A1, B1, b_rel1.reshape(1, H))

    p2 = seg128(h1, srcr, dstr, z128)
    h2 = _tc_layer(p2, h1, W_rel2.T, W_root2.T, b_rel2.reshape(1, H))

    p3 = seg128(h2, srcr, dstr, z128)
    h3 = _tc_layer(p3, h2, W_rel3.T, W_root3.T, b_rel3.reshape(1, H))

    xr = h3[:N].reshape(NG, NN, H)
    xr_p = jnp.zeros((NG, 128, H), jnp.float32).at[:, :NN, :].set(xr)
    adj_p = jnp.zeros((NG, 128, 128), jnp.float32).at[:, :NN, :NN].set(adj)
    s_p = jnp.zeros((NG, 128, C), jnp.float32).at[:, :NN, :].set(s)

    out, out_adj, acc = _tc_pool(xr_p, adj_p, s_p)
    link_loss = jnp.sqrt(acc[0, 0]) / (NG * NN * NN)
    ent_loss = acc[0, 1] / (NG * NN)
    return out, out_adj, link_loss, ent_loss, pos
